# baseline (speedup 1.0000x reference)
import functools

import jax
import jax.numpy as jnp
from jax import lax
from jax.experimental import pallas as pl
from jax.experimental.pallas import tpu as pltpu
from jax.experimental.pallas import tpu_sc as plsc

_F32 = jnp.float32
_I32 = jnp.int32

_ROW_OFF = (0, 10, 12, 16, 20)
_COL_OFF = (0, 25, 50, 75, 100, 128)
_BIAS_ROW = 28
_ZERO_ROW = 29
_N_E_ROWS = 32
_N_OUT = 40
_GROUP = 8
_ROWS_PER_WORKER = 2
_N_WORKERS = _N_OUT // _ROWS_PER_WORKER


def _project_body(idx5_ref, lvl_ref, typ_ref, fea_ref, exc_ref, par_ref,
                  w_ref, b_ref, e_ref, idx_ref):
    w = w_ref[...]
    dn = (((1,), (1,)), ((), ()))
    blocks = []
    for t_ref, k in zip((lvl_ref, typ_ref, fea_ref, exc_ref, par_ref),
                        range(5)):
        w_slice = w[:, _COL_OFF[k]:_COL_OFF[k + 1]]
        blocks.append(lax.dot_general(t_ref[...], w_slice, dn,
                                      preferred_element_type=_F32))
    blocks.append(b_ref[...])
    blocks.append(jnp.zeros((3, 128), _F32))
    e_ref[...] = jnp.concatenate(blocks, axis=0)

    k = lax.broadcasted_iota(_I32, (_N_OUT, 5), 1)
    offs = (10 * (k >= 1) + 2 * (k >= 2) + 4 * (k >= 3) + 4 * (k >= 4)
            ).astype(_I32)
    shifted = idx5_ref[...] + offs
    pad = lax.broadcasted_iota(_I32, (_N_OUT, 3), 1) + _BIAS_ROW
    idx_ref[...] = jnp.concatenate([shifted, pad], axis=1)


def _project(idx5, lvl, typ, fea, exc, par, w, b_row):
    return pl.pallas_call(
        _project_body,
        out_shape=[
            jax.ShapeDtypeStruct((_N_E_ROWS, 128), _F32),
            jax.ShapeDtypeStruct((_N_OUT, _GROUP), _I32),
        ],
    )(idx5, lvl, typ, fea, exc, par, w, b_row)


def _gather_sum_body(e_hbm, idx_hbm, out_hbm, idx_v, rows_v, out_v, sem):
    c = lax.axis_index("c")
    s = lax.axis_index("s")
    wid = s * 2 + c

    @pl.when(wid < _N_WORKERS)
    def _():
        n_idx = _ROWS_PER_WORKER * _GROUP
        pltpu.sync_copy(idx_hbm.at[pl.ds(wid * n_idx, n_idx)], idx_v)
        pltpu.async_copy(e_hbm.at[idx_v], rows_v, sem).wait()
        for j in range(128 // 16):
            sl = pl.ds(j * 16, 16)
            acc0 = rows_v[0, sl]
            acc1 = rows_v[_GROUP, sl]
            for i in range(1, _GROUP):
                acc0 = acc0 + rows_v[i, sl]
                acc1 = acc1 + rows_v[_GROUP + i, sl]
            out_v[0, sl] = acc0
            out_v[1, sl] = acc1
        pltpu.sync_copy(out_v, out_hbm.at[pl.ds(wid * _ROWS_PER_WORKER,
                                                _ROWS_PER_WORKER)])


@functools.lru_cache(maxsize=1)
def _make_gather_sum():
    return pl.kernel(
        _gather_sum_body,
        out_type=jax.ShapeDtypeStruct((_N_OUT, 128), _F32),
        mesh=plsc.VectorSubcoreMesh(core_axis_name="c", subcore_axis_name="s"),
        scratch_types=[
            pltpu.VMEM((_ROWS_PER_WORKER * _GROUP,), _I32),
            pltpu.VMEM((_ROWS_PER_WORKER * _GROUP, 128), _F32),
            pltpu.VMEM((_ROWS_PER_WORKER, 128), _F32),
            pltpu.SemaphoreType.DMA,
        ],
    )


def kernel(level_idx, type_idx, feature_idx, exchange_idx, pair_idx,
           level_table, type_table, feature_table, exchange_table, pair_table,
           W, b):
    idx5 = jnp.stack([level_idx, type_idx, feature_idx, exchange_idx,
                      pair_idx], axis=1)
    e, idx8 = _project(idx5, level_table, type_table, feature_table,
                       exchange_table, pair_table, W, b.reshape(1, 128))
    return _make_gather_sum()(e, idx8.reshape(_N_OUT * _GROUP))


# speedup vs baseline: 1.0298x; 1.0298x over previous
import functools

import jax
import jax.numpy as jnp
from jax import lax
from jax.experimental import pallas as pl
from jax.experimental.pallas import tpu as pltpu
from jax.experimental.pallas import tpu_sc as plsc

_F32 = jnp.float32
_I32 = jnp.int32

_ROW_OFF = (0, 10, 12, 16, 20)
_COL_OFF = (0, 25, 50, 75, 100, 128)
_BIAS_ROW = 28
_N_E_ROWS = 32
_N_OUT = 40
_GROUP = 8
_ROWS_PER_WORKER = 2
_N_WORKERS = _N_OUT // _ROWS_PER_WORKER


def _project_body(lvl_ref, typ_ref, fea_ref, exc_ref, par_ref, w_ref, b_ref,
                  e_ref):
    w = w_ref[...]
    dn = (((1,), (1,)), ((), ()))
    blocks = []
    for t_ref, k in zip((lvl_ref, typ_ref, fea_ref, exc_ref, par_ref),
                        range(5)):
        w_slice = w[:, _COL_OFF[k]:_COL_OFF[k + 1]]
        blocks.append(lax.dot_general(t_ref[...], w_slice, dn,
                                      preferred_element_type=_F32))
    blocks.append(jnp.reshape(b_ref[...], (1, 128)))
    blocks.append(jnp.zeros((3, 128), _F32))
    e_ref[...] = jnp.concatenate(blocks, axis=0)


def _project(lvl, typ, fea, exc, par, w, b):
    return pl.pallas_call(
        _project_body,
        out_shape=jax.ShapeDtypeStruct((_N_E_ROWS, 128), _F32),
    )(lvl, typ, fea, exc, par, w, b)


def _lane_pick(vec, idx):
    dn = lax.GatherDimensionNumbers(offset_dims=(), collapsed_slice_dims=(0,),
                                    start_index_map=(0,))
    return lax.gather(vec, idx[:, None], dn, slice_sizes=(1,),
                      mode=lax.GatherScatterMode.PROMISE_IN_BOUNDS)


def _gather_sum_body(e_hbm, i0_hbm, i1_hbm, i2_hbm, i3_hbm, i4_hbm, out_hbm,
                     i0_v, i1_v, i2_v, i3_v, i4_v, idx_v, rows_v, out_v, sem):
    c = lax.axis_index("c")
    s = lax.axis_index("s")
    wid = s * 2 + c

    @pl.when(wid < _N_WORKERS)
    def _():
        pltpu.sync_copy(i0_hbm, i0_v.at[pl.ds(0, _N_OUT)])
        pltpu.sync_copy(i1_hbm, i1_v.at[pl.ds(0, _N_OUT)])
        pltpu.sync_copy(i2_hbm, i2_v.at[pl.ds(0, _N_OUT)])
        pltpu.sync_copy(i3_hbm, i3_v.at[pl.ds(0, _N_OUT)])
        pltpu.sync_copy(i4_hbm, i4_v.at[pl.ds(0, _N_OUT)])
        lanes = lax.iota(_I32, 16)
        h = lanes >> 3
        pos = lanes & 7
        base = pl.ds(2 * wid, 16)
        val = _lane_pick(i0_v[base], h) + _ROW_OFF[0]
        for ref, off, p in ((i1_v, _ROW_OFF[1], 1), (i2_v, _ROW_OFF[2], 2),
                            (i3_v, _ROW_OFF[3], 3), (i4_v, _ROW_OFF[4], 4)):
            val = jnp.where(pos == p, _lane_pick(ref[base], h) + off, val)
        val = jnp.where(pos >= 5, pos + (_BIAS_ROW - 5), val)
        idx_v[...] = val
        pltpu.async_copy(e_hbm.at[idx_v], rows_v, sem).wait()
        for j in range(128 // 16):
            sl = pl.ds(j * 16, 16)
            acc0 = rows_v[0, sl]
            acc1 = rows_v[_GROUP, sl]
            for i in range(1, _GROUP):
                acc0 = acc0 + rows_v[i, sl]
                acc1 = acc1 + rows_v[_GROUP + i, sl]
            out_v[0, sl] = acc0
            out_v[1, sl] = acc1
        pltpu.sync_copy(out_v, out_hbm.at[pl.ds(wid * _ROWS_PER_WORKER,
                                                _ROWS_PER_WORKER)])


@functools.lru_cache(maxsize=1)
def _make_gather_sum():
    return pl.kernel(
        _gather_sum_body,
        out_type=jax.ShapeDtypeStruct((_N_OUT, 128), _F32),
        mesh=plsc.VectorSubcoreMesh(core_axis_name="c", subcore_axis_name="s"),
        scratch_types=[
            pltpu.VMEM((64,), _I32),
            pltpu.VMEM((64,), _I32),
            pltpu.VMEM((64,), _I32),
            pltpu.VMEM((64,), _I32),
            pltpu.VMEM((64,), _I32),
            pltpu.VMEM((_ROWS_PER_WORKER * _GROUP,), _I32),
            pltpu.VMEM((_ROWS_PER_WORKER * _GROUP, 128), _F32),
            pltpu.VMEM((_ROWS_PER_WORKER, 128), _F32),
            pltpu.SemaphoreType.DMA,
        ],
    )


def kernel(level_idx, type_idx, feature_idx, exchange_idx, pair_idx,
           level_table, type_table, feature_table, exchange_table, pair_table,
           W, b):
    e = _project(level_table, type_table, feature_table, exchange_table,
                 pair_table, W, b)
    return _make_gather_sum()(e, level_idx, type_idx, feature_idx,
                              exchange_idx, pair_idx)


# speedup vs baseline: 1.0764x; 1.0452x over previous
import functools

import jax
import jax.numpy as jnp
from jax import lax
from jax.experimental import pallas as pl
from jax.experimental.pallas import tpu as pltpu
from jax.experimental.pallas import tpu_sc as plsc

_F32 = jnp.float32
_I32 = jnp.int32

_ROW_OFF = (0, 10, 12, 16, 20)
_COL_OFF = (0, 25, 50, 75, 100, 128)
_BIAS_ROW = 28
_N_E_ROWS = 32
_N_OUT = 40
_IDX_PAD = 64
_GROUP = 8
_ROWS_PER_WORKER = 2
_N_WORKERS = _N_OUT // _ROWS_PER_WORKER


def _project_body(lvl_ref, typ_ref, fea_ref, exc_ref, par_ref, w_ref, b_ref,
                  i0_ref, i1_ref, i2_ref, i3_ref, i4_ref, e_ref, idxc_ref):
    w = w_ref[...]
    dn = (((1,), (1,)), ((), ()))
    blocks = []
    for t_ref, k in zip((lvl_ref, typ_ref, fea_ref, exc_ref, par_ref),
                        range(5)):
        w_slice = w[:, _COL_OFF[k]:_COL_OFF[k + 1]]
        blocks.append(lax.dot_general(t_ref[...], w_slice, dn,
                                      preferred_element_type=_F32))
    blocks.append(jnp.reshape(b_ref[...], (1, 128)))
    blocks.append(jnp.zeros((3, 128), _F32))
    e_ref[...] = jnp.concatenate(blocks, axis=0)

    pad = jnp.zeros((1, _IDX_PAD - _N_OUT), _I32)
    rows = [
        jnp.concatenate([jnp.reshape(i_ref[...] + off, (1, _N_OUT)), pad],
                        axis=1)
        for i_ref, off in zip((i0_ref, i1_ref, i2_ref, i3_ref, i4_ref),
                              _ROW_OFF)
    ]
    idxc_ref[...] = jnp.concatenate(rows, axis=0)


def _project(lvl, typ, fea, exc, par, w, b, i0, i1, i2, i3, i4):
    return pl.pallas_call(
        _project_body,
        out_shape=[
            jax.ShapeDtypeStruct((_N_E_ROWS, 128), _F32),
            jax.ShapeDtypeStruct((5, _IDX_PAD), _I32),
        ],
    )(lvl, typ, fea, exc, par, w, b, i0, i1, i2, i3, i4)


def _lane_pick(vec, idx):
    dn = lax.GatherDimensionNumbers(offset_dims=(), collapsed_slice_dims=(0,),
                                    start_index_map=(0,))
    return lax.gather(vec, idx[:, None], dn, slice_sizes=(1,),
                      mode=lax.GatherScatterMode.PROMISE_IN_BOUNDS)


def _gather_sum_body(e_hbm, idxc_hbm, out_hbm, idxc_v, idx_v, rows_v, out_v,
                     sem):
    c = lax.axis_index("c")
    s = lax.axis_index("s")
    wid = s * 2 + c

    @pl.when(wid < _N_WORKERS)
    def _():
        pltpu.sync_copy(idxc_hbm, idxc_v)
        lanes = lax.iota(_I32, 16)
        h = lanes >> 3
        pos = lanes & 7
        base = pl.ds(2 * wid, 16)
        val = _lane_pick(idxc_v[0, base], h)
        for p in range(1, 5):
            val = jnp.where(pos == p, _lane_pick(idxc_v[p, base], h), val)
        val = jnp.where(pos >= 5, pos + (_BIAS_ROW - 5), val)
        idx_v[...] = val
        pltpu.async_copy(e_hbm.at[idx_v], rows_v, sem).wait()
        for j in range(128 // 16):
            sl = pl.ds(j * 16, 16)
            acc0 = rows_v[0, sl]
            acc1 = rows_v[_GROUP, sl]
            for i in range(1, _GROUP):
                acc0 = acc0 + rows_v[i, sl]
                acc1 = acc1 + rows_v[_GROUP + i, sl]
            out_v[0, sl] = acc0
            out_v[1, sl] = acc1
        pltpu.sync_copy(out_v, out_hbm.at[pl.ds(wid * _ROWS_PER_WORKER,
                                                _ROWS_PER_WORKER)])


@functools.lru_cache(maxsize=1)
def _make_gather_sum():
    return pl.kernel(
        _gather_sum_body,
        out_type=jax.ShapeDtypeStruct((_N_OUT, 128), _F32),
        mesh=plsc.VectorSubcoreMesh(core_axis_name="c", subcore_axis_name="s"),
        scratch_types=[
            pltpu.VMEM((5, _IDX_PAD), _I32),
            pltpu.VMEM((_ROWS_PER_WORKER * _GROUP,), _I32),
            pltpu.VMEM((_ROWS_PER_WORKER * _GROUP, 128), _F32),
            pltpu.VMEM((_ROWS_PER_WORKER, 128), _F32),
            pltpu.SemaphoreType.DMA,
        ],
    )


def kernel(level_idx, type_idx, feature_idx, exchange_idx, pair_idx,
           level_table, type_table, feature_table, exchange_table, pair_table,
           W, b):
    e, idxc = _project(level_table, type_table, feature_table, exchange_table,
                       pair_table, W, b, level_idx, type_idx, feature_idx,
                       exchange_idx, pair_idx)
    return _make_gather_sum()(e, idxc)
